# Initial kernel scaffold; baseline (speedup 1.0000x reference)
import functools

import jax
import jax.numpy as jnp
from jax import lax
from jax.experimental import pallas as pl
from jax.experimental.pallas import tpu as pltpu
from jax.experimental.pallas import tpu_sc as plsc

VOCAB = 1000000
EMB = 32
B = 16384
L = 50

NC = 2
NS = 16
NW = NC * NS
BPW = B // NW
CB = 64
NCHUNK = BPW // CB
IDX = CB * L

_mesh = plsc.VectorSubcoreMesh(core_axis_name="c", subcore_axis_name="s")


@functools.partial(
    pl.kernel,
    mesh=_mesh,
    out_type=jax.ShapeDtypeStruct((B, EMB), jnp.float32),
    scratch_types=[
        pltpu.VMEM((IDX,), jnp.int32),
        pltpu.VMEM((IDX, EMB), jnp.float32),
        pltpu.VMEM((CB,), jnp.float32),
        pltpu.VMEM((CB,), jnp.float32),
        pltpu.VMEM((CB, EMB), jnp.float32),
        pltpu.SemaphoreType.DMA,
    ],
)
def _bow(table_h, labels_h, len_h, out_h, idx_v, rows_v, len_v, recip_v,
         out_v, sem):
    wid = lax.axis_index("s") * NC + lax.axis_index("c")
    base0 = wid * BPW

    def chunk(c, _):
        base = base0 + c * CB
        pltpu.sync_copy(labels_h.at[pl.ds(base * L, IDX)], idx_v)
        pltpu.async_copy(table_h.at[idx_v], rows_v, sem).wait()
        pltpu.sync_copy(len_h.at[pl.ds(base, CB)], len_v)

        def rgrp(g, _):
            len16 = len_v[pl.ds(g * 16, 16)]
            recip_v[pl.ds(g * 16, 16)] = 1.0 / len16
            return 0

        lax.fori_loop(0, CB // 16, rgrp, 0, unroll=True)

        def row(b, _):
            def tok(l, accs):
                a0, a1 = accs
                r = b * L + l
                a0 = a0 + rows_v[r, pl.ds(0, 16)]
                a1 = a1 + rows_v[r, pl.ds(16, 16)]
                return (a0, a1)

            a0, a1 = lax.fori_loop(
                0, L, tok,
                (jnp.zeros((16,), jnp.float32), jnp.zeros((16,), jnp.float32)),
                unroll=2)
            r = recip_v[b]
            out_v[b, pl.ds(0, 16)] = a0 * r
            out_v[b, pl.ds(16, 16)] = a1 * r
            return 0

        lax.fori_loop(0, CB, row, 0)
        pltpu.sync_copy(out_v, out_h.at[pl.ds(base, CB)])
        return 0

    lax.fori_loop(0, NCHUNK, chunk, 0)


def kernel(markdown_label, markdown_len, embedding_table):
    labels_flat = markdown_label.reshape(-1)
    return _bow(embedding_table, labels_flat, markdown_len)


# speedup vs baseline: 2.6923x; 2.6923x over previous
import functools

import jax
import jax.numpy as jnp
from jax import lax
from jax.experimental import pallas as pl
from jax.experimental.pallas import tpu as pltpu
from jax.experimental.pallas import tpu_sc as plsc

VOCAB = 1000000
EMB = 32
B = 16384
L = 50

NC = 2
NS = 16
NW = NC * NS
BPW = B // NW
CB = 64
NCHUNK = BPW // CB
IDX = CB * L

_mesh = plsc.VectorSubcoreMesh(core_axis_name="c", subcore_axis_name="s")


@functools.partial(
    pl.kernel,
    mesh=_mesh,
    out_type=jax.ShapeDtypeStruct((B, EMB), jnp.float32),
    scratch_types=[
        pltpu.VMEM((IDX,), jnp.int32),
        pltpu.VMEM((IDX, EMB), jnp.float32),
        pltpu.VMEM((CB,), jnp.float32),
        pltpu.VMEM((CB, EMB), jnp.float32),
        pltpu.SemaphoreType.DMA,
    ],
    compiler_params=pltpu.CompilerParams(use_tc_tiling_on_sc=False),
)
def _bow(table_h, labels_h, len_h, out_h, idx_v, rows_v, len_v,
         out_v, sem):
    wid = lax.axis_index("s") * NC + lax.axis_index("c")
    base0 = wid * BPW

    def chunk(c, _):
        base = base0 + c * CB
        pltpu.sync_copy(labels_h.at[pl.ds(base * L, IDX)], idx_v)
        pltpu.async_copy(table_h.at[idx_v], rows_v, sem).wait()
        pltpu.sync_copy(len_h.at[pl.ds(base, CB)], len_v)

        def row_grp(g, _):
            recip16 = 1.0 / len_v[pl.ds(g * 16, 16)]
            for j in range(16):
                b = g * 16 + j

                def tok(l, accs):
                    a0, a1 = accs
                    r = b * L + l
                    a0 = a0 + rows_v[r, pl.ds(0, 16)]
                    a1 = a1 + rows_v[r, pl.ds(16, 16)]
                    return (a0, a1)

                a0, a1 = lax.fori_loop(
                    0, L, tok,
                    (jnp.zeros((16,), jnp.float32),
                     jnp.zeros((16,), jnp.float32)),
                    unroll=2)
                r = recip16[j]
                out_v[b, pl.ds(0, 16)] = a0 * r
                out_v[b, pl.ds(16, 16)] = a1 * r
            return 0

        lax.fori_loop(0, CB // 16, row_grp, 0)
        pltpu.sync_copy(out_v, out_h.at[pl.ds(base, CB)])
        return 0

    lax.fori_loop(0, NCHUNK, chunk, 0)


def kernel(markdown_label, markdown_len, embedding_table):
    labels_flat = markdown_label.reshape(-1)
    return _bow(embedding_table, labels_flat, markdown_len)
